# baseline (speedup 1.0000x reference)
import functools

import jax
import jax.numpy as jnp
from jax import lax
from jax.experimental import pallas as pl
from jax.experimental.pallas import tpu as pltpu
from jax.experimental.pallas import tpu_sc as plsc

LOG2_L = 24
L_IN = 1 << LOG2_L
N_OUT = L_IN * 2
BLK = 1 << 21
NC = 2
NS = 16
NW = NC * NS
W_ELEMS = L_IN // NW
SUB_PER_BLK = BLK // W_ELEMS

NBUF = 3
DCH = 1 << 15
ZCH = 1 << 14
N_DATA_ITER = W_ELEMS // DCH
N_ZERO_ITER = W_ELEMS // ZCH


@functools.partial(
    pl.kernel,
    mesh=plsc.VectorSubcoreMesh(core_axis_name="c", subcore_axis_name="s"),
    out_type=jax.ShapeDtypeStruct((N_OUT,), jnp.float32),
    scratch_types=(
        [pltpu.VMEM((DCH,), jnp.float32) for _ in range(NBUF)]
        + [pltpu.VMEM((ZCH,), jnp.float32)]
        + [pltpu.SemaphoreType.DMA for _ in range(2 * NBUF + 1)]
    ),
)
def _add_ancilla_sc(psi_hbm, out_hbm, *scratch):
    dbufs = scratch[:NBUF]
    zbuf = scratch[NBUF]
    sem_in = scratch[NBUF + 1 : NBUF + 1 + NBUF]
    sem_out = scratch[NBUF + 1 + NBUF : NBUF + 1 + 2 * NBUF]
    sem_z = scratch[NBUF + 1 + 2 * NBUF]

    c = lax.axis_index("c")
    s = lax.axis_index("s")
    wid = c * NS + s

    h = wid // SUB_PER_BLK
    sub = wid % SUB_PER_BLK
    src_base = wid * W_ELEMS
    dst_base = h * (2 * BLK) + sub * W_ELEMS
    zdst_base = h * (2 * BLK) + BLK + sub * W_ELEMS

    def start_in(i):
        b = i % NBUF
        return pltpu.async_copy(
            psi_hbm.at[pl.ds(src_base + i * DCH, DCH)], dbufs[b], sem_in[b])

    def start_out(i):
        b = i % NBUF
        return pltpu.async_copy(
            dbufs[b], out_hbm.at[pl.ds(dst_base + i * DCH, DCH)], sem_out[b])

    in_h = {j: start_in(j) for j in range(NBUF)}

    zeros16 = jnp.zeros((16,), jnp.float32)

    def zfill(i, carry):
        zbuf[pl.ds(i * 16, 16)] = zeros16
        return carry

    lax.fori_loop(0, ZCH // 16, zfill, 0)

    Z_PER_ITER = N_ZERO_ITER // N_DATA_ITER
    out_h = {}
    z_h = []
    for i in range(N_DATA_ITER):
        in_h[i].wait()
        out_h[i] = start_out(i)
        for k in range(Z_PER_ITER):
            z = i * Z_PER_ITER + k
            z_h.append(pltpu.async_copy(
                zbuf, out_hbm.at[pl.ds(zdst_base + z * ZCH, ZCH)], sem_z))
        if i + NBUF < N_DATA_ITER:
            out_h[i].wait()
            in_h[i + NBUF] = start_in(i + NBUF)

    for i in range(N_DATA_ITER - NBUF, N_DATA_ITER):
        out_h[i].wait()
    for hz in z_h:
        hz.wait()


def kernel(psi):
    return _add_ancilla_sc(psi)


# speedup vs baseline: 1.0033x; 1.0033x over previous
import functools

import jax
import jax.numpy as jnp
from jax import lax
from jax.experimental import pallas as pl
from jax.experimental.pallas import tpu as pltpu
from jax.experimental.pallas import tpu_sc as plsc

LOG2_L = 24
L_IN = 1 << LOG2_L
N_OUT = L_IN * 2
BLK = 1 << 21
NC = 2
NS = 16
NW = NC * NS
W_ELEMS = L_IN // NW
SUB_PER_BLK = BLK // W_ELEMS

NBUF = 3
DCH = 1 << 15
ZCH = 1 << 14
N_DATA_ITER = W_ELEMS // DCH
N_ZERO_ITER = W_ELEMS // ZCH


@functools.partial(
    pl.kernel,
    mesh=plsc.VectorSubcoreMesh(core_axis_name="c", subcore_axis_name="s"),
    out_type=jax.ShapeDtypeStruct((N_OUT,), jnp.float32),
    scratch_types=(
        [pltpu.VMEM((DCH,), jnp.float32) for _ in range(NBUF)]
        + [pltpu.VMEM((ZCH,), jnp.float32)]
        + [pltpu.SemaphoreType.DMA for _ in range(2 * NBUF + 1)]
    ),
)
def _add_ancilla_sc(psi_hbm, out_hbm, *scratch):
    dbufs = scratch[:NBUF]
    zbuf = scratch[NBUF]
    sem_in = scratch[NBUF + 1 : NBUF + 1 + NBUF]
    sem_out = scratch[NBUF + 1 + NBUF : NBUF + 1 + 2 * NBUF]
    sem_z = scratch[NBUF + 1 + 2 * NBUF]

    c = lax.axis_index("c")
    s = lax.axis_index("s")
    wid = s * NC + c

    h = wid // SUB_PER_BLK
    sub = wid % SUB_PER_BLK
    src_base = wid * W_ELEMS
    dst_base = h * (2 * BLK) + sub * W_ELEMS
    zdst_base = h * (2 * BLK) + BLK + sub * W_ELEMS

    def start_in(i):
        b = i % NBUF
        return pltpu.async_copy(
            psi_hbm.at[pl.ds(src_base + i * DCH, DCH)], dbufs[b], sem_in[b])

    def start_out(i):
        b = i % NBUF
        return pltpu.async_copy(
            dbufs[b], out_hbm.at[pl.ds(dst_base + i * DCH, DCH)], sem_out[b])

    in_h = {j: start_in(j) for j in range(NBUF)}

    zeros16 = jnp.zeros((16,), jnp.float32)

    def zfill(i, carry):
        zbuf[pl.ds(i * 16, 16)] = zeros16
        return carry

    lax.fori_loop(0, ZCH // 16, zfill, 0)

    Z_PER_ITER = N_ZERO_ITER // N_DATA_ITER
    out_h = {}
    z_h = []
    for i in range(N_DATA_ITER):
        in_h[i].wait()
        out_h[i] = start_out(i)
        for k in range(Z_PER_ITER):
            z = i * Z_PER_ITER + k
            z_h.append(pltpu.async_copy(
                zbuf, out_hbm.at[pl.ds(zdst_base + z * ZCH, ZCH)], sem_z))
        if i + NBUF < N_DATA_ITER:
            out_h[i].wait()
            in_h[i + NBUF] = start_in(i + NBUF)

    for i in range(N_DATA_ITER - NBUF, N_DATA_ITER):
        out_h[i].wait()
    for hz in z_h:
        hz.wait()


def kernel(psi):
    return _add_ancilla_sc(psi)
